# Initial kernel scaffold; baseline (speedup 1.0000x reference)
import jax
import jax.numpy as jnp
from jax.experimental import pallas as pl


def kernel(node_feats, edge_feats, edge_index, W1, b1, W2, b2, gamma1, beta1, gamma2, beta2, gamma3, beta3):
    raise NotImplementedError("write your pallas kernel here")


# speedup vs baseline: 2.1749x; 2.1749x over previous
import functools

import jax
import jax.numpy as jnp
from jax import lax
from jax.experimental import pallas as pl
from jax.experimental.pallas import tpu as pltpu
from jax.experimental.pallas import tpu_sc as plsc

_N = 10000
_E = 320000
_D = 128
_DE = 16

_NC = 2
_NS = 16
_NW = _NC * _NS
_EPW = _E // _NW
_EPC = _E // _NC
_NPAD = 10240
_RPT = _NPAD // _NS
_K = 80

_mesh = plsc.VectorSubcoreMesh(core_axis_name="c", subcore_axis_name="s")


def _sc_gather_body(a_hbm, b_hbm, src_hbm, dst_hbm, g1_hbm, g2_hbm,
                    idxa, idxb, bufa, bufb, sema, semb):
    wid = lax.axis_index("s") * _NC + lax.axis_index("c")
    base = wid * _EPW

    def body(i, carry):
        off = base + i * _K
        pltpu.sync_copy(src_hbm.at[pl.ds(off, _K)], idxa)
        pltpu.sync_copy(dst_hbm.at[pl.ds(off, _K)], idxb)
        ca = pltpu.async_copy(a_hbm.at[idxa], bufa, sema)
        cb = pltpu.async_copy(b_hbm.at[idxb], bufb, semb)
        ca.wait()
        cb.wait()
        pltpu.sync_copy(bufa, g1_hbm.at[pl.ds(off, _K)])
        pltpu.sync_copy(bufb, g2_hbm.at[pl.ds(off, _K)])
        return carry

    lax.fori_loop(0, _EPW // _K, body, 0)


_sc_gather = pl.kernel(
    _sc_gather_body,
    mesh=_mesh,
    out_type=[jax.ShapeDtypeStruct((_E, _D), jnp.float32),
              jax.ShapeDtypeStruct((_E, _D), jnp.float32)],
    scratch_types=[
        pltpu.VMEM((_K,), jnp.int32),
        pltpu.VMEM((_K,), jnp.int32),
        pltpu.VMEM((_K, _D), jnp.float32),
        pltpu.VMEM((_K, _D), jnp.float32),
        pltpu.SemaphoreType.DMA,
        pltpu.SemaphoreType.DMA,
    ],
)


def _sc_scatter_body(h_hbm, dst_hbm, zeros_hbm, out_hbm, idxv, hbuf, acc, ):
    cid = lax.axis_index("c")
    sid = lax.axis_index("s")
    r0 = sid * _RPT
    pltpu.sync_copy(zeros_hbm.at[pl.ds(r0, _RPT)], acc.at[pl.ds(r0, _RPT)])
    plsc.subcore_barrier()

    ebase = cid * _EPC + sid * _EPW

    def body(i, carry):
        off = ebase + i * _K
        pltpu.sync_copy(dst_hbm.at[pl.ds(off, _K)], idxv)
        pltpu.sync_copy(h_hbm.at[pl.ds(off, _K)], hbuf)
        pltpu.sync_copy(hbuf, acc.at[idxv], add=True)
        return carry

    lax.fori_loop(0, _EPW // _K, body, 0)
    plsc.subcore_barrier()
    pltpu.sync_copy(acc.at[pl.ds(r0, _RPT)],
                    out_hbm.at[pl.ds(cid * _NPAD + r0, _RPT)])


_sc_scatter = pl.kernel(
    _sc_scatter_body,
    mesh=_mesh,
    out_type=[jax.ShapeDtypeStruct((_NC * _NPAD, _D), jnp.float32)],
    scratch_types=[
        pltpu.VMEM((_K,), jnp.int32),
        pltpu.VMEM((_K, _D), jnp.float32),
        pltpu.VMEM_SHARED((_NPAD, _D), jnp.float32),
    ],
)


def _p1_body(nf_ref, wa_ref, wb_ref, a_ref, b_ref):
    nf = nf_ref[...]
    a_ref[...] = jnp.dot(nf, wa_ref[...], preferred_element_type=jnp.float32)
    b_ref[...] = jnp.dot(nf, wb_ref[...], preferred_element_type=jnp.float32)


_P1R = 1000
_p1 = pl.pallas_call(
    _p1_body,
    grid=(_N // _P1R,),
    in_specs=[pl.BlockSpec((_P1R, _D), lambda i: (i, 0)),
              pl.BlockSpec((_D, _D), lambda i: (0, 0)),
              pl.BlockSpec((_D, _D), lambda i: (0, 0))],
    out_specs=[pl.BlockSpec((_P1R, _D), lambda i: (i, 0)),
               pl.BlockSpec((_P1R, _D), lambda i: (i, 0))],
    out_shape=[jax.ShapeDtypeStruct((_N, _D), jnp.float32),
               jax.ShapeDtypeStruct((_N, _D), jnp.float32)],
)


_R = 2000


def _p3_body(g1_ref, g2_ref, ef_ref, we_ref, b1_ref, x_ref, st_ref, acc_ref):
    x = (g1_ref[...] + g2_ref[...]
         + jnp.dot(ef_ref[...], we_ref[...], preferred_element_type=jnp.float32)
         + b1_ref[...])
    x_ref[...] = x
    s = jnp.concatenate([jnp.sum(x, axis=0, keepdims=True),
                         jnp.sum(x * x, axis=0, keepdims=True)], axis=0)

    @pl.when(pl.program_id(0) == 0)
    def _():
        acc_ref[...] = jnp.zeros_like(acc_ref)

    acc_ref[...] += s

    @pl.when(pl.program_id(0) == pl.num_programs(0) - 1)
    def _():
        st_ref[...] = acc_ref[...]


_p3 = pl.pallas_call(
    _p3_body,
    grid=(_E // _R,),
    in_specs=[pl.BlockSpec((_R, _D), lambda i: (i, 0)),
              pl.BlockSpec((_R, _D), lambda i: (i, 0)),
              pl.BlockSpec((_R, _DE), lambda i: (i, 0)),
              pl.BlockSpec((_DE, _D), lambda i: (0, 0)),
              pl.BlockSpec((1, _D), lambda i: (0, 0))],
    out_specs=[pl.BlockSpec((_R, _D), lambda i: (i, 0)),
               pl.BlockSpec((2, _D), lambda i: (0, 0))],
    out_shape=[jax.ShapeDtypeStruct((_E, _D), jnp.float32),
               jax.ShapeDtypeStruct((2, _D), jnp.float32)],
    scratch_shapes=[pltpu.VMEM((2, _D), jnp.float32)],
)


def _p4_body(x_ref, a1_ref, c1_ref, f_ref, st_ref, acc_ref):
    f = jax.nn.sigmoid(x_ref[...] * a1_ref[...] + c1_ref[...])
    f_ref[...] = f
    s = jnp.concatenate([jnp.sum(f, axis=0, keepdims=True),
                         jnp.sum(f * f, axis=0, keepdims=True)], axis=0)

    @pl.when(pl.program_id(0) == 0)
    def _():
        acc_ref[...] = jnp.zeros_like(acc_ref)

    acc_ref[...] += s

    @pl.when(pl.program_id(0) == pl.num_programs(0) - 1)
    def _():
        st_ref[...] = acc_ref[...]


_p4 = pl.pallas_call(
    _p4_body,
    grid=(_E // _R,),
    in_specs=[pl.BlockSpec((_R, _D), lambda i: (i, 0)),
              pl.BlockSpec((1, _D), lambda i: (0, 0)),
              pl.BlockSpec((1, _D), lambda i: (0, 0))],
    out_specs=[pl.BlockSpec((_R, _D), lambda i: (i, 0)),
               pl.BlockSpec((2, _D), lambda i: (0, 0))],
    out_shape=[jax.ShapeDtypeStruct((_E, _D), jnp.float32),
               jax.ShapeDtypeStruct((2, _D), jnp.float32)],
    scratch_shapes=[pltpu.VMEM((2, _D), jnp.float32)],
)


def _p5_body(f_ref, a2_ref, c2_ref, h_ref):
    f = f_ref[...]
    h_ref[...] = f * jax.nn.softplus(f * a2_ref[...] + c2_ref[...])


_p5 = pl.pallas_call(
    _p5_body,
    grid=(_E // _R,),
    in_specs=[pl.BlockSpec((_R, _D), lambda i: (i, 0)),
              pl.BlockSpec((1, _D), lambda i: (0, 0)),
              pl.BlockSpec((1, _D), lambda i: (0, 0))],
    out_specs=pl.BlockSpec((_R, _D), lambda i: (i, 0)),
    out_shape=jax.ShapeDtypeStruct((_E, _D), jnp.float32),
)


def _p7_body(p_ref, nf_ref, g3_ref, b3_ref, out_ref):
    nn = (p_ref[0] + p_ref[1])[:_N]
    m = jnp.mean(nn, axis=0, keepdims=True)
    v = jnp.mean((nn - m) ** 2, axis=0, keepdims=True)
    out_ref[...] = (nf_ref[...]
                    + g3_ref[...] * (nn - m) * lax.rsqrt(v + 1e-5)
                    + b3_ref[...])


_p7 = pl.pallas_call(
    _p7_body,
    out_shape=jax.ShapeDtypeStruct((_N, _D), jnp.float32),
)


@jax.jit
def kernel(node_feats, edge_feats, edge_index, W1, b1, W2, b2,
           gamma1, beta1, gamma2, beta2, gamma3, beta3):
    src = edge_index[0]
    dst = edge_index[1]
    wa = W1[:_D]
    wb = W1[_D:2 * _D]
    we = W1[2 * _D:]

    a_tab, b_tab = _p1(node_feats, wa, wb)
    g1, g2 = _sc_gather(a_tab, b_tab, src, dst)
    x, st1 = _p3(g1, g2, edge_feats, we, b1.reshape(1, _D))

    mean1 = st1[0] / _E
    var1 = st1[1] / _E - mean1 * mean1
    a1 = gamma1 / jnp.sqrt(var1 + 1e-5)
    c1 = beta1 - mean1 * a1

    f, st2 = _p4(x, a1.reshape(1, _D), c1.reshape(1, _D))

    mean2 = st2[0] / _E
    var2 = st2[1] / _E - mean2 * mean2
    a2 = gamma2 / jnp.sqrt(var2 + 1e-5)
    c2 = beta2 - mean2 * a2

    h = _p5(f, a2.reshape(1, _D), c2.reshape(1, _D))

    zeros = jnp.zeros((_NPAD, _D), jnp.float32)
    (partials,) = _sc_scatter(h, dst, zeros)

    out = _p7(partials.reshape(_NC, _NPAD, _D), node_feats,
              gamma3.reshape(1, _D), beta3.reshape(1, _D))
    return (out, edge_feats)


# speedup vs baseline: 2.4129x; 1.1094x over previous
import functools

import jax
import jax.numpy as jnp
from jax import lax
from jax.experimental import pallas as pl
from jax.experimental.pallas import tpu as pltpu
from jax.experimental.pallas import tpu_sc as plsc

_N = 10000
_E = 320000
_D = 128
_DE = 16

_NC = 2
_NS = 16
_NW = _NC * _NS
_EPW = _E // _NW
_EPC = _E // _NC
_NPAD = 10240
_RPT = _NPAD // _NS
_K = 80

_mesh = plsc.VectorSubcoreMesh(core_axis_name="c", subcore_axis_name="s")


def _sc_gather_body(a_hbm, b_hbm, src_hbm, dst_hbm, g1_hbm, g2_hbm,
                    idxa, idxb, bufa, bufb, sema, semb):
    wid = lax.axis_index("s") * _NC + lax.axis_index("c")
    base = wid * _EPW

    def body(i, carry):
        off = base + i * _K
        pltpu.sync_copy(src_hbm.at[pl.ds(off, _K)], idxa)
        pltpu.sync_copy(dst_hbm.at[pl.ds(off, _K)], idxb)
        ca = pltpu.async_copy(a_hbm.at[idxa], bufa, sema)
        cb = pltpu.async_copy(b_hbm.at[idxb], bufb, semb)
        ca.wait()
        cb.wait()
        pltpu.sync_copy(bufa, g1_hbm.at[pl.ds(off, _K)])
        pltpu.sync_copy(bufb, g2_hbm.at[pl.ds(off, _K)])
        return carry

    lax.fori_loop(0, _EPW // _K, body, 0)


_sc_gather = pl.kernel(
    _sc_gather_body,
    mesh=_mesh,
    out_type=[jax.ShapeDtypeStruct((_E, _D), jnp.float32),
              jax.ShapeDtypeStruct((_E, _D), jnp.float32)],
    scratch_types=[
        pltpu.VMEM((_K,), jnp.int32),
        pltpu.VMEM((_K,), jnp.int32),
        pltpu.VMEM((_K, _D), jnp.float32),
        pltpu.VMEM((_K, _D), jnp.float32),
        pltpu.SemaphoreType.DMA,
        pltpu.SemaphoreType.DMA,
    ],
)


def _sc_scatter_body(h_hbm, dst_hbm, zeros_hbm, out_hbm, idxv, hbuf, acc, ):
    cid = lax.axis_index("c")
    sid = lax.axis_index("s")
    r0 = sid * _RPT
    pltpu.sync_copy(zeros_hbm.at[pl.ds(r0, _RPT)], acc.at[pl.ds(r0, _RPT)])
    plsc.subcore_barrier()

    ebase = cid * _EPC + sid * _EPW

    def body(i, carry):
        off = ebase + i * _K
        pltpu.sync_copy(dst_hbm.at[pl.ds(off, _K)], idxv)
        pltpu.sync_copy(h_hbm.at[pl.ds(off, _K)], hbuf)
        pltpu.sync_copy(hbuf, acc.at[idxv], add=True)
        return carry

    lax.fori_loop(0, _EPW // _K, body, 0)
    plsc.subcore_barrier()
    pltpu.sync_copy(acc.at[pl.ds(r0, _RPT)],
                    out_hbm.at[pl.ds(cid * _NPAD + r0, _RPT)])


_sc_scatter = pl.kernel(
    _sc_scatter_body,
    mesh=_mesh,
    out_type=[jax.ShapeDtypeStruct((_NC * _NPAD, _D), jnp.float32)],
    scratch_types=[
        pltpu.VMEM((_K,), jnp.int32),
        pltpu.VMEM((_K, _D), jnp.float32),
        pltpu.VMEM_SHARED((_NPAD, _D), jnp.float32),
    ],
)


def _p1_body(nf_ref, wa_ref, wb_ref, a_ref, b_ref):
    nf = nf_ref[...]
    a_ref[...] = jnp.dot(nf, wa_ref[...], preferred_element_type=jnp.float32)
    b_ref[...] = jnp.dot(nf, wb_ref[...], preferred_element_type=jnp.float32)


_P1R = 1000
_p1 = pl.pallas_call(
    _p1_body,
    grid=(_N // _P1R,),
    in_specs=[pl.BlockSpec((_P1R, _D), lambda i: (i, 0)),
              pl.BlockSpec((_D, _D), lambda i: (0, 0)),
              pl.BlockSpec((_D, _D), lambda i: (0, 0))],
    out_specs=[pl.BlockSpec((_P1R, _D), lambda i: (i, 0)),
               pl.BlockSpec((_P1R, _D), lambda i: (i, 0))],
    out_shape=[jax.ShapeDtypeStruct((_N, _D), jnp.float32),
               jax.ShapeDtypeStruct((_N, _D), jnp.float32)],
)


_R = 4000


def _p3_body(g1_ref, g2_ref, ef_ref, we_ref, b1_ref, x_ref, st_ref, acc_ref):
    x = (g1_ref[...] + g2_ref[...]
         + jnp.dot(ef_ref[...], we_ref[...], preferred_element_type=jnp.float32)
         + b1_ref[...])
    x_ref[...] = x
    s = jnp.concatenate([jnp.sum(x, axis=0, keepdims=True),
                         jnp.sum(x * x, axis=0, keepdims=True)], axis=0)

    @pl.when(pl.program_id(0) == 0)
    def _():
        acc_ref[...] = jnp.zeros_like(acc_ref)

    acc_ref[...] += s

    @pl.when(pl.program_id(0) == pl.num_programs(0) - 1)
    def _():
        st_ref[...] = acc_ref[...]


_p3 = pl.pallas_call(
    _p3_body,
    grid=(_E // _R,),
    in_specs=[pl.BlockSpec((_R, _D), lambda i: (i, 0)),
              pl.BlockSpec((_R, _D), lambda i: (i, 0)),
              pl.BlockSpec((_R, _DE), lambda i: (i, 0)),
              pl.BlockSpec((_DE, _D), lambda i: (0, 0)),
              pl.BlockSpec((1, _D), lambda i: (0, 0))],
    out_specs=[pl.BlockSpec((_R, _D), lambda i: (i, 0)),
               pl.BlockSpec((2, _D), lambda i: (0, 0))],
    out_shape=[jax.ShapeDtypeStruct((_E, _D), jnp.float32),
               jax.ShapeDtypeStruct((2, _D), jnp.float32)],
    scratch_shapes=[pltpu.VMEM((2, _D), jnp.float32)],
)


def _p4_body(x_ref, a1_ref, c1_ref, st_ref, acc_ref):
    f = jax.nn.sigmoid(x_ref[...] * a1_ref[...] + c1_ref[...])
    s = jnp.concatenate([jnp.sum(f, axis=0, keepdims=True),
                         jnp.sum(f * f, axis=0, keepdims=True)], axis=0)

    @pl.when(pl.program_id(0) == 0)
    def _():
        acc_ref[...] = jnp.zeros_like(acc_ref)

    acc_ref[...] += s

    @pl.when(pl.program_id(0) == pl.num_programs(0) - 1)
    def _():
        st_ref[...] = acc_ref[...]


_p4 = pl.pallas_call(
    _p4_body,
    grid=(_E // _R,),
    in_specs=[pl.BlockSpec((_R, _D), lambda i: (i, 0)),
              pl.BlockSpec((1, _D), lambda i: (0, 0)),
              pl.BlockSpec((1, _D), lambda i: (0, 0))],
    out_specs=pl.BlockSpec((2, _D), lambda i: (0, 0)),
    out_shape=jax.ShapeDtypeStruct((2, _D), jnp.float32),
    scratch_shapes=[pltpu.VMEM((2, _D), jnp.float32)],
)


def _p5_body(x_ref, a1_ref, c1_ref, a2_ref, c2_ref, h_ref):
    f = jax.nn.sigmoid(x_ref[...] * a1_ref[...] + c1_ref[...])
    h_ref[...] = f * jax.nn.softplus(f * a2_ref[...] + c2_ref[...])


_p5 = pl.pallas_call(
    _p5_body,
    grid=(_E // _R,),
    in_specs=[pl.BlockSpec((_R, _D), lambda i: (i, 0)),
              pl.BlockSpec((1, _D), lambda i: (0, 0)),
              pl.BlockSpec((1, _D), lambda i: (0, 0)),
              pl.BlockSpec((1, _D), lambda i: (0, 0)),
              pl.BlockSpec((1, _D), lambda i: (0, 0))],
    out_specs=pl.BlockSpec((_R, _D), lambda i: (i, 0)),
    out_shape=jax.ShapeDtypeStruct((_E, _D), jnp.float32),
)


def _p7_body(p_ref, nf_ref, g3_ref, b3_ref, out_ref):
    nn = (p_ref[0] + p_ref[1])[:_N]
    m = jnp.mean(nn, axis=0, keepdims=True)
    v = jnp.mean((nn - m) ** 2, axis=0, keepdims=True)
    out_ref[...] = (nf_ref[...]
                    + g3_ref[...] * (nn - m) * lax.rsqrt(v + 1e-5)
                    + b3_ref[...])


_p7 = pl.pallas_call(
    _p7_body,
    out_shape=jax.ShapeDtypeStruct((_N, _D), jnp.float32),
)


@jax.jit
def kernel(node_feats, edge_feats, edge_index, W1, b1, W2, b2,
           gamma1, beta1, gamma2, beta2, gamma3, beta3):
    src = edge_index[0]
    dst = edge_index[1]
    wa = W1[:_D]
    wb = W1[_D:2 * _D]
    we = W1[2 * _D:]

    a_tab, b_tab = _p1(node_feats, wa, wb)
    g1, g2 = _sc_gather(a_tab, b_tab, src, dst)
    x, st1 = _p3(g1, g2, edge_feats, we, b1.reshape(1, _D))

    mean1 = st1[0] / _E
    var1 = st1[1] / _E - mean1 * mean1
    a1 = gamma1 / jnp.sqrt(var1 + 1e-5)
    c1 = beta1 - mean1 * a1

    st2 = _p4(x, a1.reshape(1, _D), c1.reshape(1, _D))

    mean2 = st2[0] / _E
    var2 = st2[1] / _E - mean2 * mean2
    a2 = gamma2 / jnp.sqrt(var2 + 1e-5)
    c2 = beta2 - mean2 * a2

    h = _p5(x, a1.reshape(1, _D), c1.reshape(1, _D),
            a2.reshape(1, _D), c2.reshape(1, _D))

    zeros = jnp.zeros((_NPAD, _D), jnp.float32)
    (partials,) = _sc_scatter(h, dst, zeros)

    out = _p7(partials.reshape(_NC, _NPAD, _D), node_feats,
              gamma3.reshape(1, _D), beta3.reshape(1, _D))
    return (out, edge_feats)


# speedup vs baseline: 2.5161x; 1.0428x over previous
import functools

import jax
import jax.numpy as jnp
from jax import lax
from jax.experimental import pallas as pl
from jax.experimental.pallas import tpu as pltpu
from jax.experimental.pallas import tpu_sc as plsc

_N = 10000
_E = 320000
_D = 128
_DE = 16

_NC = 2
_NS = 16
_NW = _NC * _NS
_EPW = _E // _NW
_EPC = _E // _NC
_NPAD = 10240
_RPT = _NPAD // _NS
_K = 80

_mesh = plsc.VectorSubcoreMesh(core_axis_name="c", subcore_axis_name="s")


def _sc_gather_body(a_hbm, b_hbm, src_hbm, dst_hbm, g1_hbm, g2_hbm,
                    idxa, idxb, bufa, bufb, sema, semb):
    wid = lax.axis_index("s") * _NC + lax.axis_index("c")
    base = wid * _EPW

    def body(i, carry):
        off = base + i * _K
        pltpu.sync_copy(src_hbm.at[pl.ds(off, _K)], idxa)
        pltpu.sync_copy(dst_hbm.at[pl.ds(off, _K)], idxb)
        ca = pltpu.async_copy(a_hbm.at[idxa], bufa, sema)
        cb = pltpu.async_copy(b_hbm.at[idxb], bufb, semb)
        ca.wait()
        cb.wait()
        pltpu.sync_copy(bufa, g1_hbm.at[pl.ds(off, _K)])
        pltpu.sync_copy(bufb, g2_hbm.at[pl.ds(off, _K)])
        return carry

    lax.fori_loop(0, _EPW // _K, body, 0)


_sc_gather = pl.kernel(
    _sc_gather_body,
    mesh=_mesh,
    out_type=[jax.ShapeDtypeStruct((_E, _D), jnp.float32),
              jax.ShapeDtypeStruct((_E, _D), jnp.float32)],
    scratch_types=[
        pltpu.VMEM((_K,), jnp.int32),
        pltpu.VMEM((_K,), jnp.int32),
        pltpu.VMEM((_K, _D), jnp.float32),
        pltpu.VMEM((_K, _D), jnp.float32),
        pltpu.SemaphoreType.DMA,
        pltpu.SemaphoreType.DMA,
    ],
)


def _sc_scatter_body(h_hbm, dst_hbm, zeros_hbm, out_hbm, idxv, hbuf, acc, ):
    cid = lax.axis_index("c")
    sid = lax.axis_index("s")
    r0 = sid * _RPT
    pltpu.sync_copy(zeros_hbm.at[pl.ds(r0, _RPT)], acc.at[pl.ds(r0, _RPT)])
    plsc.subcore_barrier()

    ebase = cid * _EPC + sid * _EPW

    def body(i, carry):
        off = ebase + i * _K
        pltpu.sync_copy(dst_hbm.at[pl.ds(off, _K)], idxv)
        pltpu.sync_copy(h_hbm.at[pl.ds(off, _K)], hbuf)
        pltpu.sync_copy(hbuf, acc.at[idxv], add=True)
        return carry

    lax.fori_loop(0, _EPW // _K, body, 0)
    plsc.subcore_barrier()
    pltpu.sync_copy(acc.at[pl.ds(r0, _RPT)],
                    out_hbm.at[pl.ds(cid * _NPAD + r0, _RPT)])


_sc_scatter = pl.kernel(
    _sc_scatter_body,
    mesh=_mesh,
    out_type=[jax.ShapeDtypeStruct((_NC * _NPAD, _D), jnp.float32)],
    scratch_types=[
        pltpu.VMEM((_K,), jnp.int32),
        pltpu.VMEM((_K, _D), jnp.float32),
        pltpu.VMEM_SHARED((_NPAD, _D), jnp.float32),
    ],
)


def _p1_body(nf_ref, wa_ref, wb_ref, a_ref, b_ref):
    nf = nf_ref[...]
    a_ref[...] = jnp.dot(nf, wa_ref[...], preferred_element_type=jnp.float32)
    b_ref[...] = jnp.dot(nf, wb_ref[...], preferred_element_type=jnp.float32)


_P1R = 1000
_p1 = pl.pallas_call(
    _p1_body,
    grid=(_N // _P1R,),
    in_specs=[pl.BlockSpec((_P1R, _D), lambda i: (i, 0)),
              pl.BlockSpec((_D, _D), lambda i: (0, 0)),
              pl.BlockSpec((_D, _D), lambda i: (0, 0))],
    out_specs=[pl.BlockSpec((_P1R, _D), lambda i: (i, 0)),
               pl.BlockSpec((_P1R, _D), lambda i: (i, 0))],
    out_shape=[jax.ShapeDtypeStruct((_N, _D), jnp.float32),
               jax.ShapeDtypeStruct((_N, _D), jnp.float32)],
)


_R = 4000


def _p3_body(g1_ref, g2_ref, ef_ref, we_ref, b1_ref, x_ref, st_ref, acc_ref):
    x = (g1_ref[...] + g2_ref[...]
         + jnp.dot(ef_ref[...], we_ref[...], preferred_element_type=jnp.float32)
         + b1_ref[...])
    x_ref[...] = x.astype(jnp.bfloat16)
    s = jnp.concatenate([jnp.sum(x, axis=0, keepdims=True),
                         jnp.sum(x * x, axis=0, keepdims=True)], axis=0)

    @pl.when(pl.program_id(0) == 0)
    def _():
        acc_ref[...] = jnp.zeros_like(acc_ref)

    acc_ref[...] += s

    @pl.when(pl.program_id(0) == pl.num_programs(0) - 1)
    def _():
        st_ref[...] = acc_ref[...]


_p3 = pl.pallas_call(
    _p3_body,
    grid=(_E // _R,),
    in_specs=[pl.BlockSpec((_R, _D), lambda i: (i, 0)),
              pl.BlockSpec((_R, _D), lambda i: (i, 0)),
              pl.BlockSpec((_R, _DE), lambda i: (i, 0)),
              pl.BlockSpec((_DE, _D), lambda i: (0, 0)),
              pl.BlockSpec((1, _D), lambda i: (0, 0))],
    out_specs=[pl.BlockSpec((_R, _D), lambda i: (i, 0)),
               pl.BlockSpec((2, _D), lambda i: (0, 0))],
    out_shape=[jax.ShapeDtypeStruct((_E, _D), jnp.bfloat16),
               jax.ShapeDtypeStruct((2, _D), jnp.float32)],
    scratch_shapes=[pltpu.VMEM((2, _D), jnp.float32)],
)


def _p4_body(x_ref, a1_ref, c1_ref, st_ref, acc_ref):
    f = jax.nn.sigmoid(x_ref[...].astype(jnp.float32) * a1_ref[...] + c1_ref[...])
    s = jnp.concatenate([jnp.sum(f, axis=0, keepdims=True),
                         jnp.sum(f * f, axis=0, keepdims=True)], axis=0)

    @pl.when(pl.program_id(0) == 0)
    def _():
        acc_ref[...] = jnp.zeros_like(acc_ref)

    acc_ref[...] += s

    @pl.when(pl.program_id(0) == pl.num_programs(0) - 1)
    def _():
        st_ref[...] = acc_ref[...]


_p4 = pl.pallas_call(
    _p4_body,
    grid=(_E // _R,),
    in_specs=[pl.BlockSpec((_R, _D), lambda i: (i, 0)),
              pl.BlockSpec((1, _D), lambda i: (0, 0)),
              pl.BlockSpec((1, _D), lambda i: (0, 0))],
    out_specs=pl.BlockSpec((2, _D), lambda i: (0, 0)),
    out_shape=jax.ShapeDtypeStruct((2, _D), jnp.float32),
    scratch_shapes=[pltpu.VMEM((2, _D), jnp.float32)],
)


def _p5_body(x_ref, a1_ref, c1_ref, a2_ref, c2_ref, h_ref):
    f = jax.nn.sigmoid(x_ref[...].astype(jnp.float32) * a1_ref[...] + c1_ref[...])
    h_ref[...] = f * jax.nn.softplus(f * a2_ref[...] + c2_ref[...])


_p5 = pl.pallas_call(
    _p5_body,
    grid=(_E // _R,),
    in_specs=[pl.BlockSpec((_R, _D), lambda i: (i, 0)),
              pl.BlockSpec((1, _D), lambda i: (0, 0)),
              pl.BlockSpec((1, _D), lambda i: (0, 0)),
              pl.BlockSpec((1, _D), lambda i: (0, 0)),
              pl.BlockSpec((1, _D), lambda i: (0, 0))],
    out_specs=pl.BlockSpec((_R, _D), lambda i: (i, 0)),
    out_shape=jax.ShapeDtypeStruct((_E, _D), jnp.float32),
)


def _p7_body(p_ref, nf_ref, g3_ref, b3_ref, out_ref):
    nn = (p_ref[0] + p_ref[1])[:_N]
    m = jnp.mean(nn, axis=0, keepdims=True)
    v = jnp.mean((nn - m) ** 2, axis=0, keepdims=True)
    out_ref[...] = (nf_ref[...]
                    + g3_ref[...] * (nn - m) * lax.rsqrt(v + 1e-5)
                    + b3_ref[...])


_p7 = pl.pallas_call(
    _p7_body,
    out_shape=jax.ShapeDtypeStruct((_N, _D), jnp.float32),
)


@jax.jit
def kernel(node_feats, edge_feats, edge_index, W1, b1, W2, b2,
           gamma1, beta1, gamma2, beta2, gamma3, beta3):
    src = edge_index[0]
    dst = edge_index[1]
    wa = W1[:_D]
    wb = W1[_D:2 * _D]
    we = W1[2 * _D:]

    a_tab, b_tab = _p1(node_feats, wa, wb)
    g1, g2 = _sc_gather(a_tab, b_tab, src, dst)
    x, st1 = _p3(g1, g2, edge_feats, we, b1.reshape(1, _D))

    mean1 = st1[0] / _E
    var1 = st1[1] / _E - mean1 * mean1
    a1 = gamma1 / jnp.sqrt(var1 + 1e-5)
    c1 = beta1 - mean1 * a1

    st2 = _p4(x, a1.reshape(1, _D), c1.reshape(1, _D))

    mean2 = st2[0] / _E
    var2 = st2[1] / _E - mean2 * mean2
    a2 = gamma2 / jnp.sqrt(var2 + 1e-5)
    c2 = beta2 - mean2 * a2

    h = _p5(x, a1.reshape(1, _D), c1.reshape(1, _D),
            a2.reshape(1, _D), c2.reshape(1, _D))

    zeros = jnp.zeros((_NPAD, _D), jnp.float32)
    (partials,) = _sc_scatter(h, dst, zeros)

    out = _p7(partials.reshape(_NC, _NPAD, _D), node_feats,
              gamma3.reshape(1, _D), beta3.reshape(1, _D))
    return (out, edge_feats)


# speedup vs baseline: 2.8446x; 1.1306x over previous
import functools

import jax
import jax.numpy as jnp
from jax import lax
from jax.experimental import pallas as pl
from jax.experimental.pallas import tpu as pltpu
from jax.experimental.pallas import tpu_sc as plsc

_N = 10000
_E = 320000
_D = 128
_DE = 16

_NC = 2
_NS = 16
_NW = _NC * _NS
_EPW = _E // _NW
_EPC = _E // _NC
_NPAD = 10240
_RPT = _NPAD // _NS
_K = 80

_mesh = plsc.VectorSubcoreMesh(core_axis_name="c", subcore_axis_name="s")


def _sc_gather_body(a_hbm, b_hbm, src_hbm, dst_hbm, g1_hbm, g2_hbm,
                    idxa, idxb, bufa, bufb, sema, semb):
    wid = lax.axis_index("s") * _NC + lax.axis_index("c")
    base = wid * _EPW

    def body(i, carry):
        off = base + i * _K
        pltpu.sync_copy(src_hbm.at[pl.ds(off, _K)], idxa)
        pltpu.sync_copy(dst_hbm.at[pl.ds(off, _K)], idxb)
        ca = pltpu.async_copy(a_hbm.at[idxa], bufa, sema)
        cb = pltpu.async_copy(b_hbm.at[idxb], bufb, semb)
        ca.wait()
        cb.wait()
        pltpu.sync_copy(bufa, g1_hbm.at[pl.ds(off, _K)])
        pltpu.sync_copy(bufb, g2_hbm.at[pl.ds(off, _K)])
        return carry

    lax.fori_loop(0, _EPW // _K, body, 0)


_sc_gather = pl.kernel(
    _sc_gather_body,
    mesh=_mesh,
    out_type=[jax.ShapeDtypeStruct((_E, _D), jnp.float32),
              jax.ShapeDtypeStruct((_E, _D), jnp.float32)],
    scratch_types=[
        pltpu.VMEM((_K,), jnp.int32),
        pltpu.VMEM((_K,), jnp.int32),
        pltpu.VMEM((_K, _D), jnp.float32),
        pltpu.VMEM((_K, _D), jnp.float32),
        pltpu.SemaphoreType.DMA,
        pltpu.SemaphoreType.DMA,
    ],
)


_NCH = _EPW // _K
_NPAIR = (_NCH - 1) // 2


def _sc_scatter_body(h_hbm, dst_hbm, zeros_hbm, out_hbm,
                     idx0, idx1, hb0, hb1, acc,
                     si0, si1, sh0, sh1):
    cid = lax.axis_index("c")
    sid = lax.axis_index("s")
    r0 = sid * _RPT
    pltpu.sync_copy(zeros_hbm.at[pl.ds(r0, _RPT)], acc.at[pl.ds(r0, _RPT)])

    ebase = cid * _EPC + sid * _EPW

    def start_loads(c, idxb, hbufb, semi, semh):
        off = ebase + c * _K
        pltpu.async_copy(dst_hbm.at[pl.ds(off, _K)], idxb, semi)
        pltpu.async_copy(h_hbm.at[pl.ds(off, _K)], hbufb, semh)

    def wait_loads(c, idxb, hbufb, semi, semh):
        off = ebase + c * _K
        pltpu.make_async_copy(dst_hbm.at[pl.ds(off, _K)], idxb, semi).wait()
        pltpu.make_async_copy(h_hbm.at[pl.ds(off, _K)], hbufb, semh).wait()

    plsc.subcore_barrier()
    start_loads(0, idx0, hb0, si0, sh0)
    start_loads(1, idx1, hb1, si1, sh1)

    def body(j, carry):
        c0 = 2 * j
        wait_loads(c0, idx0, hb0, si0, sh0)
        pltpu.sync_copy(hb0, acc.at[idx0], add=True)
        start_loads(jnp.minimum(c0 + 2, _NCH - 1), idx0, hb0, si0, sh0)
        c1 = c0 + 1
        wait_loads(c1, idx1, hb1, si1, sh1)
        pltpu.sync_copy(hb1, acc.at[idx1], add=True)
        start_loads(jnp.minimum(c1 + 2, _NCH - 1), idx1, hb1, si1, sh1)
        return carry

    lax.fori_loop(0, _NPAIR, body, 0)
    wait_loads(_NCH - 1, idx0, hb0, si0, sh0)
    wait_loads(_NCH - 1, idx1, hb1, si1, sh1)
    pltpu.sync_copy(hb0, acc.at[idx0], add=True)
    plsc.subcore_barrier()
    pltpu.sync_copy(acc.at[pl.ds(r0, _RPT)],
                    out_hbm.at[pl.ds(cid * _NPAD + r0, _RPT)])


_sc_scatter = pl.kernel(
    _sc_scatter_body,
    mesh=_mesh,
    out_type=[jax.ShapeDtypeStruct((_NC * _NPAD, _D), jnp.float32)],
    scratch_types=[
        pltpu.VMEM((_K,), jnp.int32),
        pltpu.VMEM((_K,), jnp.int32),
        pltpu.VMEM((_K, _D), jnp.float32),
        pltpu.VMEM((_K, _D), jnp.float32),
        pltpu.VMEM_SHARED((_NPAD, _D), jnp.float32),
        pltpu.SemaphoreType.DMA,
        pltpu.SemaphoreType.DMA,
        pltpu.SemaphoreType.DMA,
        pltpu.SemaphoreType.DMA,
    ],
)


def _p1_body(nf_ref, wa_ref, wb_ref, a_ref, b_ref):
    nf = nf_ref[...]
    a_ref[...] = jnp.dot(nf, wa_ref[...], preferred_element_type=jnp.float32)
    b_ref[...] = jnp.dot(nf, wb_ref[...], preferred_element_type=jnp.float32)


_P1R = 1000
_p1 = pl.pallas_call(
    _p1_body,
    grid=(_N // _P1R,),
    in_specs=[pl.BlockSpec((_P1R, _D), lambda i: (i, 0)),
              pl.BlockSpec((_D, _D), lambda i: (0, 0)),
              pl.BlockSpec((_D, _D), lambda i: (0, 0))],
    out_specs=[pl.BlockSpec((_P1R, _D), lambda i: (i, 0)),
               pl.BlockSpec((_P1R, _D), lambda i: (i, 0))],
    out_shape=[jax.ShapeDtypeStruct((_N, _D), jnp.float32),
               jax.ShapeDtypeStruct((_N, _D), jnp.float32)],
)


_R = 4000


def _p3_body(g1_ref, g2_ref, ef_ref, we_ref, b1_ref, x_ref, st_ref, acc_ref):
    x = (g1_ref[...] + g2_ref[...]
         + jnp.dot(ef_ref[...], we_ref[...], preferred_element_type=jnp.float32)
         + b1_ref[...])
    x_ref[...] = x.astype(jnp.bfloat16)
    s = jnp.concatenate([jnp.sum(x, axis=0, keepdims=True),
                         jnp.sum(x * x, axis=0, keepdims=True)], axis=0)

    @pl.when(pl.program_id(0) == 0)
    def _():
        acc_ref[...] = jnp.zeros_like(acc_ref)

    acc_ref[...] += s

    @pl.when(pl.program_id(0) == pl.num_programs(0) - 1)
    def _():
        st_ref[...] = acc_ref[...]


_p3 = pl.pallas_call(
    _p3_body,
    grid=(_E // _R,),
    in_specs=[pl.BlockSpec((_R, _D), lambda i: (i, 0)),
              pl.BlockSpec((_R, _D), lambda i: (i, 0)),
              pl.BlockSpec((_R, _DE), lambda i: (i, 0)),
              pl.BlockSpec((_DE, _D), lambda i: (0, 0)),
              pl.BlockSpec((1, _D), lambda i: (0, 0))],
    out_specs=[pl.BlockSpec((_R, _D), lambda i: (i, 0)),
               pl.BlockSpec((2, _D), lambda i: (0, 0))],
    out_shape=[jax.ShapeDtypeStruct((_E, _D), jnp.bfloat16),
               jax.ShapeDtypeStruct((2, _D), jnp.float32)],
    scratch_shapes=[pltpu.VMEM((2, _D), jnp.float32)],
)


def _p4_body(x_ref, a1_ref, c1_ref, st_ref, acc_ref):
    f = jax.nn.sigmoid(x_ref[...].astype(jnp.float32) * a1_ref[...] + c1_ref[...])
    s = jnp.concatenate([jnp.sum(f, axis=0, keepdims=True),
                         jnp.sum(f * f, axis=0, keepdims=True)], axis=0)

    @pl.when(pl.program_id(0) == 0)
    def _():
        acc_ref[...] = jnp.zeros_like(acc_ref)

    acc_ref[...] += s

    @pl.when(pl.program_id(0) == pl.num_programs(0) - 1)
    def _():
        st_ref[...] = acc_ref[...]


_p4 = pl.pallas_call(
    _p4_body,
    grid=(_E // _R,),
    in_specs=[pl.BlockSpec((_R, _D), lambda i: (i, 0)),
              pl.BlockSpec((1, _D), lambda i: (0, 0)),
              pl.BlockSpec((1, _D), lambda i: (0, 0))],
    out_specs=pl.BlockSpec((2, _D), lambda i: (0, 0)),
    out_shape=jax.ShapeDtypeStruct((2, _D), jnp.float32),
    scratch_shapes=[pltpu.VMEM((2, _D), jnp.float32)],
)


def _p5_body(x_ref, a1_ref, c1_ref, a2_ref, c2_ref, h_ref):
    f = jax.nn.sigmoid(x_ref[...].astype(jnp.float32) * a1_ref[...] + c1_ref[...])
    h_ref[...] = f * jax.nn.softplus(f * a2_ref[...] + c2_ref[...])


_p5 = pl.pallas_call(
    _p5_body,
    grid=(_E // _R,),
    in_specs=[pl.BlockSpec((_R, _D), lambda i: (i, 0)),
              pl.BlockSpec((1, _D), lambda i: (0, 0)),
              pl.BlockSpec((1, _D), lambda i: (0, 0)),
              pl.BlockSpec((1, _D), lambda i: (0, 0)),
              pl.BlockSpec((1, _D), lambda i: (0, 0))],
    out_specs=pl.BlockSpec((_R, _D), lambda i: (i, 0)),
    out_shape=jax.ShapeDtypeStruct((_E, _D), jnp.float32),
)


def _p7_body(p_ref, nf_ref, g3_ref, b3_ref, out_ref):
    nn = (p_ref[0] + p_ref[1])[:_N]
    m = jnp.mean(nn, axis=0, keepdims=True)
    v = jnp.mean((nn - m) ** 2, axis=0, keepdims=True)
    out_ref[...] = (nf_ref[...]
                    + g3_ref[...] * (nn - m) * lax.rsqrt(v + 1e-5)
                    + b3_ref[...])


_p7 = pl.pallas_call(
    _p7_body,
    out_shape=jax.ShapeDtypeStruct((_N, _D), jnp.float32),
)


@jax.jit
def kernel(node_feats, edge_feats, edge_index, W1, b1, W2, b2,
           gamma1, beta1, gamma2, beta2, gamma3, beta3):
    src = edge_index[0]
    dst = edge_index[1]
    wa = W1[:_D]
    wb = W1[_D:2 * _D]
    we = W1[2 * _D:]

    a_tab, b_tab = _p1(node_feats, wa, wb)
    g1, g2 = _sc_gather(a_tab, b_tab, src, dst)
    x, st1 = _p3(g1, g2, edge_feats, we, b1.reshape(1, _D))

    mean1 = st1[0] / _E
    var1 = st1[1] / _E - mean1 * mean1
    a1 = gamma1 / jnp.sqrt(var1 + 1e-5)
    c1 = beta1 - mean1 * a1

    st2 = _p4(x, a1.reshape(1, _D), c1.reshape(1, _D))

    mean2 = st2[0] / _E
    var2 = st2[1] / _E - mean2 * mean2
    a2 = gamma2 / jnp.sqrt(var2 + 1e-5)
    c2 = beta2 - mean2 * a2

    h = _p5(x, a1.reshape(1, _D), c1.reshape(1, _D),
            a2.reshape(1, _D), c2.reshape(1, _D))

    zeros = jnp.zeros((_NPAD, _D), jnp.float32)
    (partials,) = _sc_scatter(h, dst, zeros)

    out = _p7(partials.reshape(_NC, _NPAD, _D), node_feats,
              gamma3.reshape(1, _D), beta3.reshape(1, _D))
    return (out, edge_feats)


# speedup vs baseline: 3.6912x; 1.2976x over previous
import functools

import jax
import jax.numpy as jnp
from jax import lax
from jax.experimental import pallas as pl
from jax.experimental.pallas import tpu as pltpu
from jax.experimental.pallas import tpu_sc as plsc

_N = 10000
_E = 320000
_D = 128
_DE = 16

_NC = 2
_NS = 16
_NW = _NC * _NS
_EPW = _E // _NW
_EPC = _E // _NC
_NPAD = 10240
_RPT = _NPAD // _NS
_K = 80

_mesh = plsc.VectorSubcoreMesh(core_axis_name="c", subcore_axis_name="s")


_GNCH = _EPW // _K
_GLAST = _GNCH - 1


def _vadd_chunk(bufa, bufb, bufg):
    def row(r, carry):
        for g in range(_D // 16):
            sl = (r, pl.ds(g * 16, 16))
            bufg[sl] = bufa[sl] + bufb[sl]
        return carry

    lax.fori_loop(0, _K, row, 0)


def _sc_gather_body(a_hbm, b_hbm, src_hbm, dst_hbm, g_hbm,
                    ia0, ib0, ia1, ib1, a0, b0, g0, a1, b1, g1,
                    sia0, sib0, sia1, sib1, sga0, sgb0, sga1, sgb1, sw0, sw1):
    wid = lax.axis_index("s") * _NC + lax.axis_index("c")
    base = wid * _EPW

    sets = ((ia0, ib0, a0, b0, g0, sia0, sib0, sga0, sgb0, sw0),
            (ia1, ib1, a1, b1, g1, sia1, sib1, sga1, sgb1, sw1))

    for s in (0, 1):
        ia, ib, ba, bb, bg, sia, sib, sga, sgb, sw = sets[s]
        off = base + s * _K
        pltpu.sync_copy(src_hbm.at[pl.ds(off, _K)], ia)
        pltpu.sync_copy(dst_hbm.at[pl.ds(off, _K)], ib)
        pltpu.async_copy(a_hbm.at[ia], ba, sga)
        pltpu.async_copy(b_hbm.at[ib], bb, sgb)

    def body(j, carry):
        for s in (0, 1):
            ia, ib, ba, bb, bg, sia, sib, sga, sgb, sw = sets[s]
            c = 2 * j + s
            off = base + c * _K
            cn = jnp.minimum(c + 2, _GLAST)
            offn = base + cn * _K
            pltpu.make_async_copy(a_hbm.at[ia], ba, sga).wait()
            pltpu.make_async_copy(b_hbm.at[ib], bb, sgb).wait()
            pltpu.async_copy(src_hbm.at[pl.ds(offn, _K)], ia, sia)
            pltpu.async_copy(dst_hbm.at[pl.ds(offn, _K)], ib, sib)

            @pl.when(c >= 2)
            def _():
                offp = base + (c - 2) * _K
                pltpu.make_async_copy(bg, g_hbm.at[pl.ds(offp, _K)], sw).wait()

            _vadd_chunk(ba, bb, bg)
            pltpu.async_copy(bg, g_hbm.at[pl.ds(off, _K)], sw)
            pltpu.make_async_copy(src_hbm.at[pl.ds(offn, _K)], ia, sia).wait()
            pltpu.make_async_copy(dst_hbm.at[pl.ds(offn, _K)], ib, sib).wait()
            pltpu.async_copy(a_hbm.at[ia], ba, sga)
            pltpu.async_copy(b_hbm.at[ib], bb, sgb)
        return carry

    lax.fori_loop(0, (_GNCH - 1) // 2, body, 0)

    ia, ib, ba, bb, bg, sia, sib, sga, sgb, sw = sets[0]
    pltpu.make_async_copy(a_hbm.at[ia], ba, sga).wait()
    pltpu.make_async_copy(b_hbm.at[ib], bb, sgb).wait()
    offp = base + (_GLAST - 2) * _K
    pltpu.make_async_copy(bg, g_hbm.at[pl.ds(offp, _K)], sw).wait()
    _vadd_chunk(ba, bb, bg)
    off = base + _GLAST * _K
    pltpu.sync_copy(bg, g_hbm.at[pl.ds(off, _K)])

    ia, ib, ba, bb, bg, sia, sib, sga, sgb, sw = sets[1]
    pltpu.make_async_copy(a_hbm.at[ia], ba, sga).wait()
    pltpu.make_async_copy(b_hbm.at[ib], bb, sgb).wait()
    offp = base + (_GLAST - 1) * _K
    pltpu.make_async_copy(bg, g_hbm.at[pl.ds(offp, _K)], sw).wait()


_sc_gather = pl.kernel(
    _sc_gather_body,
    mesh=_mesh,
    out_type=[jax.ShapeDtypeStruct((_E, _D), jnp.float32)],
    scratch_types=[
        pltpu.VMEM((_K,), jnp.int32),
        pltpu.VMEM((_K,), jnp.int32),
        pltpu.VMEM((_K,), jnp.int32),
        pltpu.VMEM((_K,), jnp.int32),
        pltpu.VMEM((_K, _D), jnp.float32),
        pltpu.VMEM((_K, _D), jnp.float32),
        pltpu.VMEM((_K, _D), jnp.float32),
        pltpu.VMEM((_K, _D), jnp.float32),
        pltpu.VMEM((_K, _D), jnp.float32),
        pltpu.VMEM((_K, _D), jnp.float32),
        pltpu.SemaphoreType.DMA,
        pltpu.SemaphoreType.DMA,
        pltpu.SemaphoreType.DMA,
        pltpu.SemaphoreType.DMA,
        pltpu.SemaphoreType.DMA,
        pltpu.SemaphoreType.DMA,
        pltpu.SemaphoreType.DMA,
        pltpu.SemaphoreType.DMA,
        pltpu.SemaphoreType.DMA,
        pltpu.SemaphoreType.DMA,
    ],
)


_NCH = _EPW // _K
_NPAIR = (_NCH - 1) // 2


def _sc_scatter_body(h_hbm, dst_hbm, zeros_hbm, out_hbm,
                     idx0, idx1, hb0, hb1, acc,
                     si0, si1, sh0, sh1):
    cid = lax.axis_index("c")
    sid = lax.axis_index("s")
    r0 = sid * _RPT
    pltpu.sync_copy(zeros_hbm.at[pl.ds(r0, _RPT)], acc.at[pl.ds(r0, _RPT)])

    ebase = cid * _EPC + sid * _EPW

    def start_loads(c, idxb, hbufb, semi, semh):
        off = ebase + c * _K
        pltpu.async_copy(dst_hbm.at[pl.ds(off, _K)], idxb, semi)
        pltpu.async_copy(h_hbm.at[pl.ds(off, _K)], hbufb, semh)

    def wait_loads(c, idxb, hbufb, semi, semh):
        off = ebase + c * _K
        pltpu.make_async_copy(dst_hbm.at[pl.ds(off, _K)], idxb, semi).wait()
        pltpu.make_async_copy(h_hbm.at[pl.ds(off, _K)], hbufb, semh).wait()

    plsc.subcore_barrier()
    start_loads(0, idx0, hb0, si0, sh0)
    start_loads(1, idx1, hb1, si1, sh1)

    def body(j, carry):
        c0 = 2 * j
        wait_loads(c0, idx0, hb0, si0, sh0)
        pltpu.sync_copy(hb0, acc.at[idx0], add=True)
        start_loads(jnp.minimum(c0 + 2, _NCH - 1), idx0, hb0, si0, sh0)
        c1 = c0 + 1
        wait_loads(c1, idx1, hb1, si1, sh1)
        pltpu.sync_copy(hb1, acc.at[idx1], add=True)
        start_loads(jnp.minimum(c1 + 2, _NCH - 1), idx1, hb1, si1, sh1)
        return carry

    lax.fori_loop(0, _NPAIR, body, 0)
    wait_loads(_NCH - 1, idx0, hb0, si0, sh0)
    wait_loads(_NCH - 1, idx1, hb1, si1, sh1)
    pltpu.sync_copy(hb0, acc.at[idx0], add=True)
    plsc.subcore_barrier()
    pltpu.sync_copy(acc.at[pl.ds(r0, _RPT)],
                    out_hbm.at[pl.ds(cid * _NPAD + r0, _RPT)])


_sc_scatter = pl.kernel(
    _sc_scatter_body,
    mesh=_mesh,
    out_type=[jax.ShapeDtypeStruct((_NC * _NPAD, _D), jnp.float32)],
    scratch_types=[
        pltpu.VMEM((_K,), jnp.int32),
        pltpu.VMEM((_K,), jnp.int32),
        pltpu.VMEM((_K, _D), jnp.float32),
        pltpu.VMEM((_K, _D), jnp.float32),
        pltpu.VMEM_SHARED((_NPAD, _D), jnp.float32),
        pltpu.SemaphoreType.DMA,
        pltpu.SemaphoreType.DMA,
        pltpu.SemaphoreType.DMA,
        pltpu.SemaphoreType.DMA,
    ],
)


def _p1_body(nf_ref, wa_ref, wb_ref, a_ref, b_ref):
    nf = nf_ref[...]
    a_ref[...] = jnp.dot(nf, wa_ref[...], preferred_element_type=jnp.float32)
    b_ref[...] = jnp.dot(nf, wb_ref[...], preferred_element_type=jnp.float32)


_P1R = 1000
_p1 = pl.pallas_call(
    _p1_body,
    grid=(_N // _P1R,),
    in_specs=[pl.BlockSpec((_P1R, _D), lambda i: (i, 0)),
              pl.BlockSpec((_D, _D), lambda i: (0, 0)),
              pl.BlockSpec((_D, _D), lambda i: (0, 0))],
    out_specs=[pl.BlockSpec((_P1R, _D), lambda i: (i, 0)),
               pl.BlockSpec((_P1R, _D), lambda i: (i, 0))],
    out_shape=[jax.ShapeDtypeStruct((_N, _D), jnp.float32),
               jax.ShapeDtypeStruct((_N, _D), jnp.float32)],
)


_R = 4000


def _p3_body(g_ref, ef_ref, we_ref, b1_ref, x_ref, st_ref, acc_ref):
    x = (g_ref[...]
         + jnp.dot(ef_ref[...], we_ref[...], preferred_element_type=jnp.float32)
         + b1_ref[...])
    x_ref[...] = x.astype(jnp.bfloat16)
    s = jnp.concatenate([jnp.sum(x, axis=0, keepdims=True),
                         jnp.sum(x * x, axis=0, keepdims=True)], axis=0)

    @pl.when(pl.program_id(0) == 0)
    def _():
        acc_ref[...] = jnp.zeros_like(acc_ref)

    acc_ref[...] += s

    @pl.when(pl.program_id(0) == pl.num_programs(0) - 1)
    def _():
        st_ref[...] = acc_ref[...]


_p3 = pl.pallas_call(
    _p3_body,
    grid=(_E // _R,),
    in_specs=[pl.BlockSpec((_R, _D), lambda i: (i, 0)),
              pl.BlockSpec((_R, _DE), lambda i: (i, 0)),
              pl.BlockSpec((_DE, _D), lambda i: (0, 0)),
              pl.BlockSpec((1, _D), lambda i: (0, 0))],
    out_specs=[pl.BlockSpec((_R, _D), lambda i: (i, 0)),
               pl.BlockSpec((2, _D), lambda i: (0, 0))],
    out_shape=[jax.ShapeDtypeStruct((_E, _D), jnp.bfloat16),
               jax.ShapeDtypeStruct((2, _D), jnp.float32)],
    scratch_shapes=[pltpu.VMEM((2, _D), jnp.float32)],
)


def _p4_body(x_ref, a1_ref, c1_ref, st_ref, acc_ref):
    f = jax.nn.sigmoid(x_ref[...].astype(jnp.float32) * a1_ref[...] + c1_ref[...])
    s = jnp.concatenate([jnp.sum(f, axis=0, keepdims=True),
                         jnp.sum(f * f, axis=0, keepdims=True)], axis=0)

    @pl.when(pl.program_id(0) == 0)
    def _():
        acc_ref[...] = jnp.zeros_like(acc_ref)

    acc_ref[...] += s

    @pl.when(pl.program_id(0) == pl.num_programs(0) - 1)
    def _():
        st_ref[...] = acc_ref[...]


_p4 = pl.pallas_call(
    _p4_body,
    grid=(_E // _R,),
    in_specs=[pl.BlockSpec((_R, _D), lambda i: (i, 0)),
              pl.BlockSpec((1, _D), lambda i: (0, 0)),
              pl.BlockSpec((1, _D), lambda i: (0, 0))],
    out_specs=pl.BlockSpec((2, _D), lambda i: (0, 0)),
    out_shape=jax.ShapeDtypeStruct((2, _D), jnp.float32),
    scratch_shapes=[pltpu.VMEM((2, _D), jnp.float32)],
)


def _p5_body(x_ref, a1_ref, c1_ref, a2_ref, c2_ref, h_ref):
    f = jax.nn.sigmoid(x_ref[...].astype(jnp.float32) * a1_ref[...] + c1_ref[...])
    h_ref[...] = f * jax.nn.softplus(f * a2_ref[...] + c2_ref[...])


_p5 = pl.pallas_call(
    _p5_body,
    grid=(_E // _R,),
    in_specs=[pl.BlockSpec((_R, _D), lambda i: (i, 0)),
              pl.BlockSpec((1, _D), lambda i: (0, 0)),
              pl.BlockSpec((1, _D), lambda i: (0, 0)),
              pl.BlockSpec((1, _D), lambda i: (0, 0)),
              pl.BlockSpec((1, _D), lambda i: (0, 0))],
    out_specs=pl.BlockSpec((_R, _D), lambda i: (i, 0)),
    out_shape=jax.ShapeDtypeStruct((_E, _D), jnp.float32),
)


def _p7_body(p_ref, nf_ref, g3_ref, b3_ref, out_ref):
    nn = (p_ref[0] + p_ref[1])[:_N]
    m = jnp.mean(nn, axis=0, keepdims=True)
    v = jnp.mean((nn - m) ** 2, axis=0, keepdims=True)
    out_ref[...] = (nf_ref[...]
                    + g3_ref[...] * (nn - m) * lax.rsqrt(v + 1e-5)
                    + b3_ref[...])


_p7 = pl.pallas_call(
    _p7_body,
    out_shape=jax.ShapeDtypeStruct((_N, _D), jnp.float32),
)


@jax.jit
def kernel(node_feats, edge_feats, edge_index, W1, b1, W2, b2,
           gamma1, beta1, gamma2, beta2, gamma3, beta3):
    src = edge_index[0]
    dst = edge_index[1]
    wa = W1[:_D]
    wb = W1[_D:2 * _D]
    we = W1[2 * _D:]

    a_tab, b_tab = _p1(node_feats, wa, wb)
    (g,) = _sc_gather(a_tab, b_tab, src, dst)
    x, st1 = _p3(g, edge_feats, we, b1.reshape(1, _D))

    mean1 = st1[0] / _E
    var1 = st1[1] / _E - mean1 * mean1
    a1 = gamma1 / jnp.sqrt(var1 + 1e-5)
    c1 = beta1 - mean1 * a1

    st2 = _p4(x, a1.reshape(1, _D), c1.reshape(1, _D))

    mean2 = st2[0] / _E
    var2 = st2[1] / _E - mean2 * mean2
    a2 = gamma2 / jnp.sqrt(var2 + 1e-5)
    c2 = beta2 - mean2 * a2

    h = _p5(x, a1.reshape(1, _D), c1.reshape(1, _D),
            a2.reshape(1, _D), c2.reshape(1, _D))

    zeros = jnp.zeros((_NPAD, _D), jnp.float32)
    (partials,) = _sc_scatter(h, dst, zeros)

    out = _p7(partials.reshape(_NC, _NPAD, _D), node_feats,
              gamma3.reshape(1, _D), beta3.reshape(1, _D))
    return (out, edge_feats)


# speedup vs baseline: 3.9127x; 1.0600x over previous
import functools

import jax
import jax.numpy as jnp
from jax import lax
from jax.experimental import pallas as pl
from jax.experimental.pallas import tpu as pltpu
from jax.experimental.pallas import tpu_sc as plsc

_N = 10000
_E = 320000
_D = 128
_DE = 16

_NC = 2
_NS = 16
_NW = _NC * _NS
_EPW = _E // _NW
_EPC = _E // _NC
_NPAD = 10240
_RPT = _NPAD // _NS
_K = 80

_mesh = plsc.VectorSubcoreMesh(core_axis_name="c", subcore_axis_name="s")


_GNCH = _EPW // _K
_GLAST = _GNCH - 1


def _vadd_chunk(bufa, bufb, bufg):
    def row(r, carry):
        for g in range(_D // 16):
            sl = (r, pl.ds(g * 16, 16))
            bufg[sl] = bufa[sl] + bufb[sl]
        return carry

    lax.fori_loop(0, _K, row, 0)


def _sc_gather_body(a_hbm, b_hbm, src_hbm, dst_hbm, g_hbm,
                    ia0, ib0, ia1, ib1, a0, b0, g0, a1, b1, g1,
                    sia0, sib0, sia1, sib1, sga0, sgb0, sga1, sgb1, sw0, sw1):
    wid = lax.axis_index("s") * _NC + lax.axis_index("c")
    base = wid * _EPW

    sets = ((ia0, ib0, a0, b0, g0, sia0, sib0, sga0, sgb0, sw0),
            (ia1, ib1, a1, b1, g1, sia1, sib1, sga1, sgb1, sw1))

    for s in (0, 1):
        ia, ib, ba, bb, bg, sia, sib, sga, sgb, sw = sets[s]
        off = base + s * _K
        pltpu.sync_copy(src_hbm.at[pl.ds(off, _K)], ia)
        pltpu.sync_copy(dst_hbm.at[pl.ds(off, _K)], ib)
        pltpu.async_copy(a_hbm.at[ia], ba, sga)
        pltpu.async_copy(b_hbm.at[ib], bb, sgb)

    def body(j, carry):
        for s in (0, 1):
            ia, ib, ba, bb, bg, sia, sib, sga, sgb, sw = sets[s]
            c = 2 * j + s
            off = base + c * _K
            cn = jnp.minimum(c + 2, _GLAST)
            offn = base + cn * _K
            pltpu.make_async_copy(a_hbm.at[ia], ba, sga).wait()
            pltpu.make_async_copy(b_hbm.at[ib], bb, sgb).wait()
            pltpu.async_copy(src_hbm.at[pl.ds(offn, _K)], ia, sia)
            pltpu.async_copy(dst_hbm.at[pl.ds(offn, _K)], ib, sib)

            @pl.when(c >= 2)
            def _():
                offp = base + (c - 2) * _K
                pltpu.make_async_copy(bg, g_hbm.at[pl.ds(offp, _K)], sw).wait()

            _vadd_chunk(ba, bb, bg)
            pltpu.async_copy(bg, g_hbm.at[pl.ds(off, _K)], sw)
            pltpu.make_async_copy(src_hbm.at[pl.ds(offn, _K)], ia, sia).wait()
            pltpu.make_async_copy(dst_hbm.at[pl.ds(offn, _K)], ib, sib).wait()
            pltpu.async_copy(a_hbm.at[ia], ba, sga)
            pltpu.async_copy(b_hbm.at[ib], bb, sgb)
        return carry

    lax.fori_loop(0, (_GNCH - 1) // 2, body, 0)

    ia, ib, ba, bb, bg, sia, sib, sga, sgb, sw = sets[0]
    pltpu.make_async_copy(a_hbm.at[ia], ba, sga).wait()
    pltpu.make_async_copy(b_hbm.at[ib], bb, sgb).wait()
    offp = base + (_GLAST - 2) * _K
    pltpu.make_async_copy(bg, g_hbm.at[pl.ds(offp, _K)], sw).wait()
    _vadd_chunk(ba, bb, bg)
    off = base + _GLAST * _K
    pltpu.sync_copy(bg, g_hbm.at[pl.ds(off, _K)])

    ia, ib, ba, bb, bg, sia, sib, sga, sgb, sw = sets[1]
    pltpu.make_async_copy(a_hbm.at[ia], ba, sga).wait()
    pltpu.make_async_copy(b_hbm.at[ib], bb, sgb).wait()
    offp = base + (_GLAST - 1) * _K
    pltpu.make_async_copy(bg, g_hbm.at[pl.ds(offp, _K)], sw).wait()


_sc_gather = pl.kernel(
    _sc_gather_body,
    mesh=_mesh,
    out_type=[jax.ShapeDtypeStruct((_E, _D), jnp.float32)],
    scratch_types=[
        pltpu.VMEM((_K,), jnp.int32),
        pltpu.VMEM((_K,), jnp.int32),
        pltpu.VMEM((_K,), jnp.int32),
        pltpu.VMEM((_K,), jnp.int32),
        pltpu.VMEM((_K, _D), jnp.float32),
        pltpu.VMEM((_K, _D), jnp.float32),
        pltpu.VMEM((_K, _D), jnp.float32),
        pltpu.VMEM((_K, _D), jnp.float32),
        pltpu.VMEM((_K, _D), jnp.float32),
        pltpu.VMEM((_K, _D), jnp.float32),
        pltpu.SemaphoreType.DMA,
        pltpu.SemaphoreType.DMA,
        pltpu.SemaphoreType.DMA,
        pltpu.SemaphoreType.DMA,
        pltpu.SemaphoreType.DMA,
        pltpu.SemaphoreType.DMA,
        pltpu.SemaphoreType.DMA,
        pltpu.SemaphoreType.DMA,
        pltpu.SemaphoreType.DMA,
        pltpu.SemaphoreType.DMA,
    ],
)


_NCH = _EPW // _K
_NPAIR = (_NCH - 1) // 2


def _sc_scatter_body(h_hbm, dst_hbm, zeros_hbm, out_hbm,
                     idx0, idx1, hb0, hb1, acc,
                     si0, si1, sh0, sh1):
    cid = lax.axis_index("c")
    sid = lax.axis_index("s")
    r0 = sid * _RPT
    pltpu.sync_copy(zeros_hbm.at[pl.ds(r0, _RPT)], acc.at[pl.ds(r0, _RPT)])

    ebase = cid * _EPC + sid * _EPW

    def start_loads(c, idxb, hbufb, semi, semh):
        off = ebase + c * _K
        pltpu.async_copy(dst_hbm.at[pl.ds(off, _K)], idxb, semi)
        pltpu.async_copy(h_hbm.at[pl.ds(off, _K)], hbufb, semh)

    def wait_loads(c, idxb, hbufb, semi, semh):
        off = ebase + c * _K
        pltpu.make_async_copy(dst_hbm.at[pl.ds(off, _K)], idxb, semi).wait()
        pltpu.make_async_copy(h_hbm.at[pl.ds(off, _K)], hbufb, semh).wait()

    plsc.subcore_barrier()
    start_loads(0, idx0, hb0, si0, sh0)
    start_loads(1, idx1, hb1, si1, sh1)

    def body(j, carry):
        c0 = 2 * j
        wait_loads(c0, idx0, hb0, si0, sh0)
        pltpu.sync_copy(hb0, acc.at[idx0], add=True)
        start_loads(jnp.minimum(c0 + 2, _NCH - 1), idx0, hb0, si0, sh0)
        c1 = c0 + 1
        wait_loads(c1, idx1, hb1, si1, sh1)
        pltpu.sync_copy(hb1, acc.at[idx1], add=True)
        start_loads(jnp.minimum(c1 + 2, _NCH - 1), idx1, hb1, si1, sh1)
        return carry

    lax.fori_loop(0, _NPAIR, body, 0)
    wait_loads(_NCH - 1, idx0, hb0, si0, sh0)
    wait_loads(_NCH - 1, idx1, hb1, si1, sh1)
    pltpu.sync_copy(hb0, acc.at[idx0], add=True)
    plsc.subcore_barrier()
    pltpu.sync_copy(acc.at[pl.ds(r0, _RPT)],
                    out_hbm.at[pl.ds(cid * _NPAD + r0, _RPT)])


_sc_scatter = pl.kernel(
    _sc_scatter_body,
    mesh=_mesh,
    out_type=[jax.ShapeDtypeStruct((_NC * _NPAD, _D), jnp.float32)],
    scratch_types=[
        pltpu.VMEM((_K,), jnp.int32),
        pltpu.VMEM((_K,), jnp.int32),
        pltpu.VMEM((_K, _D), jnp.float32),
        pltpu.VMEM((_K, _D), jnp.float32),
        pltpu.VMEM_SHARED((_NPAD, _D), jnp.float32),
        pltpu.SemaphoreType.DMA,
        pltpu.SemaphoreType.DMA,
        pltpu.SemaphoreType.DMA,
        pltpu.SemaphoreType.DMA,
    ],
)


def _p1_body(nf_ref, wa_ref, wb_ref, a_ref, b_ref):
    nf = nf_ref[...]
    a_ref[...] = jnp.dot(nf, wa_ref[...], preferred_element_type=jnp.float32)
    b_ref[...] = jnp.dot(nf, wb_ref[...], preferred_element_type=jnp.float32)


_P1R = 10000
_p1 = pl.pallas_call(
    _p1_body,
    grid=(_N // _P1R,),
    in_specs=[pl.BlockSpec((_P1R, _D), lambda i: (i, 0)),
              pl.BlockSpec((_D, _D), lambda i: (0, 0)),
              pl.BlockSpec((_D, _D), lambda i: (0, 0))],
    out_specs=[pl.BlockSpec((_P1R, _D), lambda i: (i, 0)),
               pl.BlockSpec((_P1R, _D), lambda i: (i, 0))],
    out_shape=[jax.ShapeDtypeStruct((_N, _D), jnp.float32),
               jax.ShapeDtypeStruct((_N, _D), jnp.float32)],
)


_R = 8000


def _p3_body(g_ref, ef_ref, we_ref, b1_ref, x_ref, st_ref, acc_ref):
    x = (g_ref[...]
         + jnp.dot(ef_ref[...], we_ref[...], preferred_element_type=jnp.float32)
         + b1_ref[...])
    x_ref[...] = x.astype(jnp.bfloat16)
    s = jnp.concatenate([jnp.sum(x, axis=0, keepdims=True),
                         jnp.sum(x * x, axis=0, keepdims=True)], axis=0)

    @pl.when(pl.program_id(0) == 0)
    def _():
        acc_ref[...] = jnp.zeros_like(acc_ref)

    acc_ref[...] += s

    @pl.when(pl.program_id(0) == pl.num_programs(0) - 1)
    def _():
        st_ref[...] = acc_ref[...]


_p3 = pl.pallas_call(
    _p3_body,
    grid=(_E // _R,),
    in_specs=[pl.BlockSpec((_R, _D), lambda i: (i, 0)),
              pl.BlockSpec((_R, _DE), lambda i: (i, 0)),
              pl.BlockSpec((_DE, _D), lambda i: (0, 0)),
              pl.BlockSpec((1, _D), lambda i: (0, 0))],
    out_specs=[pl.BlockSpec((_R, _D), lambda i: (i, 0)),
               pl.BlockSpec((2, _D), lambda i: (0, 0))],
    out_shape=[jax.ShapeDtypeStruct((_E, _D), jnp.bfloat16),
               jax.ShapeDtypeStruct((2, _D), jnp.float32)],
    scratch_shapes=[pltpu.VMEM((2, _D), jnp.float32)],
)


def _bn_coeffs(st, gamma, beta):
    m = st[0:1] / _E
    v = st[1:2] / _E - m * m
    a = gamma * lax.rsqrt(v + 1e-5)
    return a, beta - m * a


def _p4_body(x_ref, st1_ref, g1c_ref, b1c_ref, st_ref, acc_ref):
    a1, c1 = _bn_coeffs(st1_ref[...], g1c_ref[...], b1c_ref[...])
    f = jax.nn.sigmoid(x_ref[...].astype(jnp.float32) * a1 + c1)
    s = jnp.concatenate([jnp.sum(f, axis=0, keepdims=True),
                         jnp.sum(f * f, axis=0, keepdims=True)], axis=0)

    @pl.when(pl.program_id(0) == 0)
    def _():
        acc_ref[...] = jnp.zeros_like(acc_ref)

    acc_ref[...] += s

    @pl.when(pl.program_id(0) == pl.num_programs(0) - 1)
    def _():
        st_ref[...] = acc_ref[...]


_p4 = pl.pallas_call(
    _p4_body,
    grid=(_E // _R,),
    in_specs=[pl.BlockSpec((_R, _D), lambda i: (i, 0)),
              pl.BlockSpec((2, _D), lambda i: (0, 0)),
              pl.BlockSpec((1, _D), lambda i: (0, 0)),
              pl.BlockSpec((1, _D), lambda i: (0, 0))],
    out_specs=pl.BlockSpec((2, _D), lambda i: (0, 0)),
    out_shape=jax.ShapeDtypeStruct((2, _D), jnp.float32),
    scratch_shapes=[pltpu.VMEM((2, _D), jnp.float32)],
)


def _p5_body(x_ref, st1_ref, g1c_ref, b1c_ref, st2_ref, g2c_ref, b2c_ref,
             h_ref):
    a1, c1 = _bn_coeffs(st1_ref[...], g1c_ref[...], b1c_ref[...])
    a2, c2 = _bn_coeffs(st2_ref[...], g2c_ref[...], b2c_ref[...])
    f = jax.nn.sigmoid(x_ref[...].astype(jnp.float32) * a1 + c1)
    h_ref[...] = f * jax.nn.softplus(f * a2 + c2)


_p5 = pl.pallas_call(
    _p5_body,
    grid=(_E // _R,),
    in_specs=[pl.BlockSpec((_R, _D), lambda i: (i, 0)),
              pl.BlockSpec((2, _D), lambda i: (0, 0)),
              pl.BlockSpec((1, _D), lambda i: (0, 0)),
              pl.BlockSpec((1, _D), lambda i: (0, 0)),
              pl.BlockSpec((2, _D), lambda i: (0, 0)),
              pl.BlockSpec((1, _D), lambda i: (0, 0)),
              pl.BlockSpec((1, _D), lambda i: (0, 0))],
    out_specs=pl.BlockSpec((_R, _D), lambda i: (i, 0)),
    out_shape=jax.ShapeDtypeStruct((_E, _D), jnp.float32),
)


def _p7_body(p_ref, nf_ref, g3_ref, b3_ref, out_ref):
    nn = (p_ref[0] + p_ref[1])[:_N]
    m = jnp.mean(nn, axis=0, keepdims=True)
    v = jnp.mean((nn - m) ** 2, axis=0, keepdims=True)
    out_ref[...] = (nf_ref[...]
                    + g3_ref[...] * (nn - m) * lax.rsqrt(v + 1e-5)
                    + b3_ref[...])


_p7 = pl.pallas_call(
    _p7_body,
    out_shape=jax.ShapeDtypeStruct((_N, _D), jnp.float32),
)


@jax.jit
def kernel(node_feats, edge_feats, edge_index, W1, b1, W2, b2,
           gamma1, beta1, gamma2, beta2, gamma3, beta3):
    src = edge_index[0]
    dst = edge_index[1]
    wa = W1[:_D]
    wb = W1[_D:2 * _D]
    we = W1[2 * _D:]

    a_tab, b_tab = _p1(node_feats, wa, wb)
    (g,) = _sc_gather(a_tab, b_tab, src, dst)
    x, st1 = _p3(g, edge_feats, we, b1.reshape(1, _D))

    g1c = gamma1.reshape(1, _D)
    b1c = beta1.reshape(1, _D)
    st2 = _p4(x, st1, g1c, b1c)
    h = _p5(x, st1, g1c, b1c, st2, gamma2.reshape(1, _D),
            beta2.reshape(1, _D))

    zeros = jnp.zeros((_NPAD, _D), jnp.float32)
    (partials,) = _sc_scatter(h, dst, zeros)

    out = _p7(partials.reshape(_NC, _NPAD, _D), node_feats,
              gamma3.reshape(1, _D), beta3.reshape(1, _D))
    return (out, edge_feats)
